# baseline (speedup 1.0000x reference)
import jax
import jax.numpy as jnp
from jax import lax
from jax.experimental import pallas as pl
from jax.experimental.pallas import tpu as pltpu
from jax.experimental.pallas import tpu_sc as plsc

NC = 2
NS = 16
L = 16

B = 4096
K = 200
D = 64
N = 100000
VH = N // NC
KPAD = 208
NV = KPAD // L
PER_T = B // NS
BLK = 56
NBLK = 4
CW = BLK * NBLK + 64
VSTG = VH // NS
OB = 8

EPS = 1e-08
NEG = float(jnp.finfo(jnp.float32).min)


def _sc_body(v_hbm, comb_hbm, out_hbm,
             vsh, comb_v, cidx_v, cw_v, rows_v, out_v, cnt_s, sem0, sem1):
    cid = lax.axis_index("c")
    sid = lax.axis_index("s")
    base = sid * PER_T
    lo_f = (cid * VH).astype(jnp.float32)
    hi_f = ((cid + 1) * VH).astype(jnp.float32)
    sems = (sem0, sem1)

    pltpu.sync_copy(v_hbm.at[pl.ds(cid * VH + sid * VSTG, VSTG)],
                    vsh.at[pl.ds(sid * VSTG, VSTG)])
    plsc.subcore_barrier()

    def gather_descr(buf, b):
        return pltpu.make_async_copy(
            v_hbm.at[cidx_v.at[buf, pl.ds(b * BLK, BLK)]],
            rows_v.at[buf, pl.ds(b * BLK, BLK)],
            sems[buf])

    def lane_reduce(vec, op):
        lane = lax.iota(jnp.int32, L)
        for shift in (8, 4, 2, 1):
            idx = (lane + shift) & (L - 1)
            rot = vec.at[idx].get(mode="promise_in_bounds")
            vec = op(vec, rot)
        return vec

    def softmax_compact_fire(g, buf):
        pltpu.sync_copy(comb_hbm.at[pl.ds(3 * (base + g), 3)], comb_v)
        lo_i = cid * VH
        ivecs = [plsc.bitcast(comb_v[0, pl.ds(L * k, L)], jnp.int32)
                 for k in range(NV)]
        svs = [comb_v[1, pl.ds(L * k, L)] for k in range(NV)]
        mvs = [comb_v[2, pl.ds(L * k, L)] for k in range(NV)]
        bms = [mv > 0 for mv in mvs]
        masked = [jnp.where(bm, sv, NEG) for sv, bm in zip(svs, bms)]
        mx = masked[0]
        for t in masked[1:]:
            mx = jnp.maximum(mx, t)
        rmax = lane_reduce(mx, jnp.maximum)
        rmax = jnp.where(rmax == NEG, jnp.zeros((L,), jnp.float32), rmax)
        es = [jnp.exp(sv - rmax) * mv for sv, mv in zip(svs, mvs)]
        tot = es[0]
        for t in es[1:]:
            tot = tot + t
        denom = jnp.maximum(lane_reduce(tot, jnp.add), EPS)
        inv = jnp.float32(1) / denom
        cnt = jnp.int32(0)
        for k in range(NV):
            ivf = ivecs[k].astype(jnp.float32)
            keep = bms[k] & (ivf >= lo_f) & (ivf < hi_f)
            keep_f = jnp.where(keep, jnp.float32(1), jnp.float32(0))
            plsc.store_compressed(cw_v.at[buf, pl.ds(cnt, L)],
                                  es[k] * inv, mask=keep)
            plsc.store_compressed(cidx_v.at[buf, pl.ds(cnt, L)],
                                  ivecs[k], mask=keep)
            cnt = cnt + jnp.sum(keep_f).astype(jnp.int32)
        zf = jnp.zeros((L,), jnp.float32)
        zi = jnp.zeros((L,), jnp.int32)
        for t in range(4):
            cw_v[buf, pl.ds(cnt + t * L, L)] = zf
            cidx_v[buf, pl.ds(cnt + t * L, L)] = zi
        cnt_s[buf] = cnt
        for b in range(NBLK):
            @pl.when(cnt > b * BLK)
            def _():
                gather_descr(buf, b).start()

    def drain(buf):
        cnt = cnt_s[buf]
        for b in range(NBLK):
            @pl.when(cnt > b * BLK)
            def _():
                gather_descr(buf, b).wait()

    def accumulate(buf, g):
        cnt = cnt_s[buf]
        nv = (cnt + (L - 1)) // L

        def body_k(k, accs):
            a0, a1, a2, a3 = accs
            wvec = cw_v[buf, pl.ds(k * L, L)]
            for j in range(L):
                wb = wvec.at[jnp.full((L,), j, jnp.int32)].get(
                    mode="promise_in_bounds")
                r = k * L + j
                lop = rows_v[buf, r, pl.ds(0, 2 * L)]
                hip = rows_v[buf, r, pl.ds(2 * L, 2 * L)]
                r0, r1 = plsc.unpack(lop, format=plsc.PackFormat.INTERLEAVED)
                r2, r3 = plsc.unpack(hip, format=plsc.PackFormat.INTERLEAVED)
                a0 = a0 + wb * r0
                a1 = a1 + wb * r1
                a2 = a2 + wb * r2
                a3 = a3 + wb * r3
            return a0, a1, a2, a3
        z = jnp.zeros((L,), jnp.float32)
        a0, a1, a2, a3 = lax.fori_loop(0, nv, body_k, (z, z, z, z))
        out_v[0, pl.ds(0, L)] = a0
        out_v[0, pl.ds(L, L)] = a1
        out_v[0, pl.ds(2 * L, L)] = a2
        out_v[0, pl.ds(3 * L, L)] = a3
        pltpu.sync_copy(out_v.at[pl.ds(0, 1)],
                        out_hbm.at[pl.ds(cid * B + base + g, 1)])

    def phase(g, cur, nxt):
        @pl.when(g + 1 < PER_T)
        def _():
            softmax_compact_fire(g + 1, nxt)
        drain(cur)
        accumulate(cur, g)

    softmax_compact_fire(0, 0)

    def loop_body(p, _):
        phase(2 * p, 0, 1)
        phase(2 * p + 1, 1, 0)
        return _

    lax.fori_loop(0, PER_T // 2, loop_body, None)


@jax.jit
def _sc_call(v, comb):
    mesh = plsc.VectorSubcoreMesh(core_axis_name="c", subcore_axis_name="s")
    kern = pl.kernel(
        _sc_body,
        out_type=jax.ShapeDtypeStruct((NC * B, D), jnp.float32),
        mesh=mesh,
        scratch_types=[
            pltpu.VMEM_SHARED((VH, D), jnp.bfloat16),
            pltpu.VMEM((3, KPAD), jnp.float32),
            pltpu.VMEM((2, CW), jnp.int32),
            pltpu.VMEM((2, CW), jnp.float32),
            pltpu.VMEM((2, BLK * NBLK, D), jnp.bfloat16),
            pltpu.VMEM((OB, D), jnp.float32),
            pltpu.SMEM((2,), jnp.int32),
            pltpu.SemaphoreType.DMA,
            pltpu.SemaphoreType.DMA,
        ],
        compiler_params=pltpu.CompilerParams(
            use_tc_tiling_on_sc=False, needs_layout_passes=False),
    )
    return kern(v, comb)


def kernel(v, batch_idx, mask, count, rank_scores):
    del count
    pad = ((0, 0), (0, KPAD - K))
    idx_f = jax.lax.bitcast_convert_type(
        jnp.pad(batch_idx.astype(jnp.int32), pad), jnp.float32)
    m = jnp.pad(mask.astype(jnp.float32), pad)
    s = jnp.pad(rank_scores.astype(jnp.float32), pad)
    comb = jnp.stack([idx_f, s, m], axis=1).reshape(3 * B, KPAD)
    n = v.shape[0]
    vperm = (v.astype(jnp.float32).reshape(n, 2, 2, L)
             .transpose(0, 1, 3, 2).reshape(n, D).astype(jnp.bfloat16))
    parts = _sc_call(vperm, comb).reshape(NC, B, D)
    return parts[0] + parts[1]


# speedup vs baseline: 3.1037x; 3.1037x over previous
import jax
import jax.numpy as jnp
from jax import lax
from jax.experimental import pallas as pl
from jax.experimental.pallas import tpu as pltpu
from jax.experimental.pallas import tpu_sc as plsc

NC = 2
NS = 16
L = 16

B = 4096
K = 200
D = 64
N = 100000
VH = N // NC
KPAD = 208
NV = KPAD // L
PER_T = B // NS
BLK = 56
NBLK = 4
CW = BLK * NBLK + 64
VSTG = VH // NS
OB = 8

EPS = 1e-08
NEG = float(jnp.finfo(jnp.float32).min)


def _sc_body(v_hbm, comb_hbm, out_hbm,
             vsh, comb_v, cidx_v, cw_v, rows_v, out_v, cnt_s, sem0, sem1):
    cid = lax.axis_index("c")
    sid = lax.axis_index("s")
    base = sid * PER_T
    lo_f = (cid * VH).astype(jnp.float32)
    hi_f = ((cid + 1) * VH).astype(jnp.float32)
    sems = (sem0, sem1)

    pltpu.sync_copy(v_hbm.at[pl.ds(cid * VH + sid * VSTG, VSTG)],
                    vsh.at[pl.ds(sid * VSTG, VSTG)])
    plsc.subcore_barrier()

    def gather_descr(buf, b):
        return pltpu.make_async_copy(
            vsh.at[cidx_v.at[buf, pl.ds(b * BLK, BLK)]],
            rows_v.at[buf, pl.ds(b * BLK, BLK)],
            sems[buf])

    def lane_reduce(vec, op):
        lane = lax.iota(jnp.int32, L)
        for shift in (8, 4, 2, 1):
            idx = (lane + shift) & (L - 1)
            rot = vec.at[idx].get(mode="promise_in_bounds")
            vec = op(vec, rot)
        return vec

    def softmax_compact_fire(g, buf):
        pltpu.sync_copy(comb_hbm.at[pl.ds(3 * (base + g), 3)], comb_v)
        lo_i = cid * VH
        ivecs = [plsc.bitcast(comb_v[0, pl.ds(L * k, L)], jnp.int32)
                 for k in range(NV)]
        svs = [comb_v[1, pl.ds(L * k, L)] for k in range(NV)]
        mvs = [comb_v[2, pl.ds(L * k, L)] for k in range(NV)]
        bms = [mv > 0 for mv in mvs]
        masked = [jnp.where(bm, sv, NEG) for sv, bm in zip(svs, bms)]
        mx = masked[0]
        for t in masked[1:]:
            mx = jnp.maximum(mx, t)
        rmax = lane_reduce(mx, jnp.maximum)
        rmax = jnp.where(rmax == NEG, jnp.zeros((L,), jnp.float32), rmax)
        es = [jnp.exp(sv - rmax) * mv for sv, mv in zip(svs, mvs)]
        tot = es[0]
        for t in es[1:]:
            tot = tot + t
        denom = jnp.maximum(lane_reduce(tot, jnp.add), EPS)
        inv = jnp.float32(1) / denom
        cnt = jnp.int32(0)
        for k in range(NV):
            ivf = ivecs[k].astype(jnp.float32)
            keep = bms[k] & (ivf >= lo_f) & (ivf < hi_f)
            keep_f = jnp.where(keep, jnp.float32(1), jnp.float32(0))
            plsc.store_compressed(cw_v.at[buf, pl.ds(cnt, L)],
                                  es[k] * inv, mask=keep)
            plsc.store_compressed(cidx_v.at[buf, pl.ds(cnt, L)],
                                  ivecs[k] - lo_i, mask=keep)
            cnt = cnt + jnp.sum(keep_f).astype(jnp.int32)
        zf = jnp.zeros((L,), jnp.float32)
        zi = jnp.zeros((L,), jnp.int32)
        for t in range(4):
            cw_v[buf, pl.ds(cnt + t * L, L)] = zf
            cidx_v[buf, pl.ds(cnt + t * L, L)] = zi
        cnt_s[buf] = cnt
        for b in range(NBLK):
            @pl.when(cnt > b * BLK)
            def _():
                gather_descr(buf, b).start()

    def drain(buf):
        cnt = cnt_s[buf]
        for b in range(NBLK):
            @pl.when(cnt > b * BLK)
            def _():
                gather_descr(buf, b).wait()

    def accumulate(buf, g):
        cnt = cnt_s[buf]
        nv = (cnt + (L - 1)) // L

        def body_k(k, accs):
            a0, a1, a2, a3 = accs
            wvec = cw_v[buf, pl.ds(k * L, L)]
            for j in range(L):
                wb = wvec.at[jnp.full((L,), j, jnp.int32)].get(
                    mode="promise_in_bounds")
                r = k * L + j
                lop = rows_v[buf, r, pl.ds(0, 2 * L)]
                hip = rows_v[buf, r, pl.ds(2 * L, 2 * L)]
                r0, r1 = plsc.unpack(lop, format=plsc.PackFormat.INTERLEAVED)
                r2, r3 = plsc.unpack(hip, format=plsc.PackFormat.INTERLEAVED)
                a0 = a0 + wb * r0
                a1 = a1 + wb * r1
                a2 = a2 + wb * r2
                a3 = a3 + wb * r3
            return a0, a1, a2, a3
        z = jnp.zeros((L,), jnp.float32)
        a0, a1, a2, a3 = lax.fori_loop(0, nv, body_k, (z, z, z, z))
        out_v[0, pl.ds(0, L)] = a0
        out_v[0, pl.ds(L, L)] = a1
        out_v[0, pl.ds(2 * L, L)] = a2
        out_v[0, pl.ds(3 * L, L)] = a3
        pltpu.sync_copy(out_v.at[pl.ds(0, 1)],
                        out_hbm.at[pl.ds(cid * B + base + g, 1)])

    def phase(g, cur, nxt):
        @pl.when(g + 1 < PER_T)
        def _():
            softmax_compact_fire(g + 1, nxt)
        drain(cur)
        accumulate(cur, g)

    softmax_compact_fire(0, 0)

    def loop_body(p, _):
        phase(2 * p, 0, 1)
        phase(2 * p + 1, 1, 0)
        return _

    lax.fori_loop(0, PER_T // 2, loop_body, None)


@jax.jit
def _sc_call(v, comb):
    mesh = plsc.VectorSubcoreMesh(core_axis_name="c", subcore_axis_name="s")
    kern = pl.kernel(
        _sc_body,
        out_type=jax.ShapeDtypeStruct((NC * B, D), jnp.float32),
        mesh=mesh,
        scratch_types=[
            pltpu.VMEM_SHARED((VH, D), jnp.bfloat16),
            pltpu.VMEM((3, KPAD), jnp.float32),
            pltpu.VMEM((2, CW), jnp.int32),
            pltpu.VMEM((2, CW), jnp.float32),
            pltpu.VMEM((2, BLK * NBLK, D), jnp.bfloat16),
            pltpu.VMEM((OB, D), jnp.float32),
            pltpu.SMEM((2,), jnp.int32),
            pltpu.SemaphoreType.DMA,
            pltpu.SemaphoreType.DMA,
        ],
        compiler_params=pltpu.CompilerParams(
            use_tc_tiling_on_sc=False, needs_layout_passes=False),
    )
    return kern(v, comb)


def kernel(v, batch_idx, mask, count, rank_scores):
    del count
    pad = ((0, 0), (0, KPAD - K))
    idx_f = jax.lax.bitcast_convert_type(
        jnp.pad(batch_idx.astype(jnp.int32), pad), jnp.float32)
    m = jnp.pad(mask.astype(jnp.float32), pad)
    s = jnp.pad(rank_scores.astype(jnp.float32), pad)
    comb = jnp.stack([idx_f, s, m], axis=1).reshape(3 * B, KPAD)
    n = v.shape[0]
    vperm = (v.astype(jnp.float32).reshape(n, 2, 2, L)
             .transpose(0, 1, 3, 2).reshape(n, D).astype(jnp.bfloat16))
    parts = _sc_call(vperm, comb).reshape(NC, B, D)
    return parts[0] + parts[1]
